# baseline (speedup 1.0000x reference)
import functools

import jax
import jax.numpy as jnp
from jax import lax
from jax.experimental import pallas as pl
from jax.experimental.pallas import tpu as pltpu
from jax.experimental.pallas import tpu_sc as plsc

_B = 16384
_F = 128
_L = 16
_C = 128


def _mf_body(u_idx, i_idx, u_emb, i_emb, u_bias, i_bias, out,
             uidx_v, iidx_v, urows_v, irows_v, ub_v, ib_v, dots_v, out_v, sem,
             *, rows_per_worker, num_cores):
    wid = lax.axis_index("s") * num_cores + lax.axis_index("c")
    lane_iota = lax.iota(jnp.int32, _L)

    for c in range(rows_per_worker // _C):
        base = wid * rows_per_worker + c * _C
        pltpu.sync_copy(u_idx.at[pl.ds(base, _C)], uidx_v)
        pltpu.sync_copy(i_idx.at[pl.ds(base, _C)], iidx_v)
        cps = [
            pltpu.async_copy(u_emb.at[uidx_v], urows_v, sem),
            pltpu.async_copy(i_emb.at[iidx_v], irows_v, sem),
            pltpu.async_copy(u_bias.at[uidx_v], ub_v, sem),
            pltpu.async_copy(i_bias.at[iidx_v], ib_v, sem),
        ]
        for cp in cps:
            cp.wait()

        def zero_body(g, _):
            dots_v[pl.ds(g * _L, _L)] = jnp.zeros((_L,), jnp.float32)
            return 0

        lax.fori_loop(0, _C // _L, zero_body, 0)

        def row_group_body(g, _):
            gbase = g * _L
            for r in range(_L):
                row = gbase + r
                acc = urows_v[row, pl.ds(0, _L)] * irows_v[row, pl.ds(0, _L)]
                for k in range(1, _F // _L):
                    acc += (urows_v[row, pl.ds(k * _L, _L)]
                            * irows_v[row, pl.ds(k * _L, _L)])
                plsc.addupdate_scatter(dots_v, [jnp.full((_L,), row, jnp.int32)],
                                       acc)
            gslice = pl.ds(gbase, _L)
            gidx = jnp.full((_L,), gbase, jnp.int32) + lane_iota
            ub = plsc.load_gather(ub_v, [gidx, jnp.zeros((_L,), jnp.int32)])
            ib = plsc.load_gather(ib_v, [gidx, jnp.zeros((_L,), jnp.int32)])
            pred = dots_v[gslice] + ub + ib
            out_v[gslice] = 1.0 / (1.0 + jnp.exp(-pred))
            return 0

        lax.fori_loop(0, _C // _L, row_group_body, 0)
        pltpu.sync_copy(out_v, out.at[pl.ds(base, _C)])


@functools.cache
def _build():
    info = plsc.get_sparse_core_info()
    num_workers = info.num_cores * info.num_subcores
    rows_per_worker = _B // num_workers
    mesh = plsc.VectorSubcoreMesh(core_axis_name="c", subcore_axis_name="s")
    body = functools.partial(_mf_body, rows_per_worker=rows_per_worker,
                             num_cores=info.num_cores)
    return pl.kernel(
        body,
        out_type=jax.ShapeDtypeStruct((_B,), jnp.float32),
        mesh=mesh,
        compiler_params=pltpu.CompilerParams(needs_layout_passes=False, use_tc_tiling_on_sc=False),
        scratch_types=[
            pltpu.VMEM((_C,), jnp.int32),
            pltpu.VMEM((_C,), jnp.int32),
            pltpu.VMEM((_C, _F), jnp.float32),
            pltpu.VMEM((_C, _F), jnp.float32),
            pltpu.VMEM((_C, 1), jnp.float32),
            pltpu.VMEM((_C, 1), jnp.float32),
            pltpu.VMEM((_C,), jnp.float32),
            pltpu.VMEM((_C,), jnp.float32),
            pltpu.SemaphoreType.DMA,
        ],
    )


def kernel(u_idx, i_idx, u_emb, i_emb, u_bias, i_bias):
    return _build()(u_idx.astype(jnp.int32), i_idx.astype(jnp.int32),
                    u_emb, i_emb, u_bias, i_bias)


# speedup vs baseline: 13.7759x; 13.7759x over previous
import functools

import jax
import jax.numpy as jnp
from jax import lax
from jax.experimental import pallas as pl
from jax.experimental.pallas import tpu as pltpu
from jax.experimental.pallas import tpu_sc as plsc

_B = 16384
_F = 128
_L = 16
_C = 128
_NBUF = 2


def _dots_body(u_idx, i_idx, u_emb, i_emb, dots,
               uidx_v, iidx_v, urows_v, irows_v, dots_v, sems,
               *, rows_per_worker, num_cores):
    wid = lax.axis_index("s") * num_cores + lax.axis_index("c")
    nchunks = rows_per_worker // _C

    def start_chunk(c, b):
        base = wid * rows_per_worker + c * _C
        pltpu.sync_copy(u_idx.at[pl.ds(base, _C)], uidx_v.at[b])
        pltpu.sync_copy(i_idx.at[pl.ds(base, _C)], iidx_v.at[b])
        return [
            pltpu.async_copy(u_emb.at[uidx_v.at[b]], urows_v.at[b], sems.at[b]),
            pltpu.async_copy(i_emb.at[iidx_v.at[b]], irows_v.at[b], sems.at[b]),
        ]

    cps = {}
    for b in range(_NBUF):
        cps[b] = start_chunk(b, b)

    for c in range(nchunks):
        b = c % _NBUF
        for cp in cps[b]:
            cp.wait()

        def zero_body(g, _):
            dots_v[pl.ds(g * _L, _L)] = jnp.zeros((_L,), jnp.float32)
            return 0

        lax.fori_loop(0, _C // _L, zero_body, 0)

        uv = urows_v.at[b]
        iv = irows_v.at[b]

        def row_group_body(g, _):
            gbase = g * _L
            for r in range(_L):
                row = gbase + r
                acc = uv[row, pl.ds(0, _L)] * iv[row, pl.ds(0, _L)]
                for k in range(1, _F // _L):
                    acc += (uv[row, pl.ds(k * _L, _L)]
                            * iv[row, pl.ds(k * _L, _L)])
                plsc.addupdate_scatter(dots_v,
                                       [jnp.full((_L,), row, jnp.int32)], acc)
            return 0

        lax.fori_loop(0, _C // _L, row_group_body, 0)
        base = wid * rows_per_worker + c * _C
        pltpu.sync_copy(dots_v, dots.at[pl.ds(base, _C)])
        if c + _NBUF < nchunks:
            cps[b] = start_chunk(c + _NBUF, b)


def _bias_body(u_idx, i_idx, dots, u_bias, i_bias, out,
               uidx_v, iidx_v, ub_v, ib_v, dots_v, sem,
               *, rows_per_worker, num_cores):
    wid = lax.axis_index("s") * num_cores + lax.axis_index("c")

    for c in range(rows_per_worker // _C):
        base = wid * rows_per_worker + c * _C
        pltpu.sync_copy(u_idx.at[pl.ds(base, _C)], uidx_v)
        pltpu.sync_copy(i_idx.at[pl.ds(base, _C)], iidx_v)
        pltpu.sync_copy(dots.at[pl.ds(base, _C)], dots_v)
        cps = [
            pltpu.async_copy(u_bias.at[uidx_v], ub_v, sem),
            pltpu.async_copy(i_bias.at[iidx_v], ib_v, sem),
        ]
        for cp in cps:
            cp.wait()

        def group_body(g, _):
            gslice = pl.ds(g * _L, _L)
            pred = dots_v[gslice] + ub_v[gslice] + ib_v[gslice]
            dots_v[gslice] = 1.0 / (1.0 + jnp.exp(-pred))
            return 0

        lax.fori_loop(0, _C // _L, group_body, 0)
        pltpu.sync_copy(dots_v, out.at[pl.ds(base, _C)])


@functools.cache
def _build():
    info = plsc.get_sparse_core_info()
    num_workers = info.num_cores * info.num_subcores
    rpw = _B // num_workers
    mesh = plsc.VectorSubcoreMesh(core_axis_name="c", subcore_axis_name="s")
    params = pltpu.CompilerParams(needs_layout_passes=False)

    dots_k = pl.kernel(
        functools.partial(_dots_body, rows_per_worker=rpw,
                          num_cores=info.num_cores),
        out_type=jax.ShapeDtypeStruct((_B,), jnp.float32),
        mesh=mesh,
        compiler_params=params,
        scratch_types=[
            pltpu.VMEM((_NBUF, _C), jnp.int32),
            pltpu.VMEM((_NBUF, _C), jnp.int32),
            pltpu.VMEM((_NBUF, _C, _F), jnp.float32),
            pltpu.VMEM((_NBUF, _C, _F), jnp.float32),
            pltpu.VMEM((_C,), jnp.float32),
            pltpu.SemaphoreType.DMA((_NBUF,)),
        ],
    )
    bias_k = pl.kernel(
        functools.partial(_bias_body, rows_per_worker=rpw,
                          num_cores=info.num_cores),
        out_type=jax.ShapeDtypeStruct((_B,), jnp.float32),
        mesh=mesh,
        compiler_params=params,
        scratch_types=[
            pltpu.VMEM((_C,), jnp.int32),
            pltpu.VMEM((_C,), jnp.int32),
            pltpu.VMEM((_C,), jnp.float32),
            pltpu.VMEM((_C,), jnp.float32),
            pltpu.VMEM((_C,), jnp.float32),
            pltpu.SemaphoreType.DMA,
        ],
    )
    return dots_k, bias_k


def kernel(u_idx, i_idx, u_emb, i_emb, u_bias, i_bias):
    dots_k, bias_k = _build()
    ui = u_idx.astype(jnp.int32)
    ii = i_idx.astype(jnp.int32)
    dots = dots_k(ui, ii, u_emb, i_emb)
    return bias_k(ui, ii, dots, u_bias.reshape(-1), i_bias.reshape(-1))


# speedup vs baseline: 14.7807x; 1.0729x over previous
import functools

import jax
import jax.numpy as jnp
from jax import lax
from jax.experimental import pallas as pl
from jax.experimental.pallas import tpu as pltpu
from jax.experimental.pallas import tpu_sc as plsc

_B = 16384
_F = 128
_L = 16
_C = 128
_NBUF = 2


def _dots_body(u_idx, i_idx, u_emb, i_emb, dots,
               uidx_v, iidx_v, urows_v, irows_v, dots_v, sems,
               *, rows_per_worker, num_cores):
    wid = lax.axis_index("s") * num_cores + lax.axis_index("c")
    nchunks = rows_per_worker // _C

    def start_chunk(c, b):
        base = wid * rows_per_worker + c * _C
        pltpu.sync_copy(u_idx.at[pl.ds(base, _C)], uidx_v.at[b])
        pltpu.sync_copy(i_idx.at[pl.ds(base, _C)], iidx_v.at[b])
        return [
            pltpu.async_copy(u_emb.at[uidx_v.at[b]], urows_v.at[b], sems.at[b]),
            pltpu.async_copy(i_emb.at[iidx_v.at[b]], irows_v.at[b], sems.at[b]),
        ]

    cps = {}
    for b in range(_NBUF):
        cps[b] = start_chunk(b, b)

    for c in range(nchunks):
        b = c % _NBUF
        for cp in cps[b]:
            cp.wait()

        def zero_body(g, _):
            dots_v[pl.ds(g * _L, _L)] = jnp.zeros((_L,), jnp.float32)
            return 0

        lax.fori_loop(0, _C // _L, zero_body, 0)

        uv = urows_v.at[b]
        iv = irows_v.at[b]

        def row_group_body(g, _):
            gbase = g * _L
            for r in range(_L):
                row = gbase + r
                acc = uv[row, pl.ds(0, _L)] * iv[row, pl.ds(0, _L)]
                for k in range(1, _F // _L):
                    acc += (uv[row, pl.ds(k * _L, _L)]
                            * iv[row, pl.ds(k * _L, _L)])
                plsc.addupdate_scatter(dots_v,
                                       [jnp.full((_L,), row, jnp.int32)], acc)
            return 0

        lax.fori_loop(0, _C // _L, row_group_body, 0)
        base = wid * rows_per_worker + c * _C
        pltpu.sync_copy(dots_v, dots.at[pl.ds(base, _C)])
        if c + _NBUF < nchunks:
            cps[b] = start_chunk(c + _NBUF, b)


def _bias_body(u_idx, i_idx, dots, u_bias, i_bias, out,
               uidx_v, iidx_v, ub_v, ib_v, dots_v, sem_i, sem_d,
               *, rows_per_worker, num_cores):
    wid = lax.axis_index("s") * num_cores + lax.axis_index("c")
    nchunks = rows_per_worker // _C

    def stage(c, b):
        base = wid * rows_per_worker + c * _C
        return [
            pltpu.async_copy(u_idx.at[pl.ds(base, _C)], uidx_v.at[b],
                             sem_i.at[b]),
            pltpu.async_copy(i_idx.at[pl.ds(base, _C)], iidx_v.at[b],
                             sem_i.at[b]),
        ], pltpu.async_copy(dots.at[pl.ds(base, _C)], dots_v.at[b],
                            sem_d.at[b])

    staged = {}
    for b in range(_NBUF):
        staged[b] = stage(b, b)

    for c in range(nchunks):
        b = c % _NBUF
        idx_cps, dots_cp = staged[b]
        for cp in idx_cps:
            cp.wait()
        gath = [
            pltpu.async_copy(u_bias.at[uidx_v.at[b]], ub_v.at[b], sem_d.at[b]),
            pltpu.async_copy(i_bias.at[iidx_v.at[b]], ib_v.at[b], sem_d.at[b]),
        ]
        dots_cp.wait()
        for cp in gath:
            cp.wait()

        dv = dots_v.at[b]
        ubv = ub_v.at[b]
        ibv = ib_v.at[b]

        def group_body(g, _):
            gslice = pl.ds(g * _L, _L)
            pred = dv[gslice] + ubv[gslice] + ibv[gslice]
            dv[gslice] = 1.0 / (1.0 + jnp.exp(-pred))
            return 0

        lax.fori_loop(0, _C // _L, group_body, 0)
        base = wid * rows_per_worker + c * _C
        pltpu.sync_copy(dots_v.at[b], out.at[pl.ds(base, _C)])
        if c + _NBUF < nchunks:
            staged[b] = stage(c + _NBUF, b)


@functools.cache
def _build():
    info = plsc.get_sparse_core_info()
    num_workers = info.num_cores * info.num_subcores
    rpw = _B // num_workers
    mesh = plsc.VectorSubcoreMesh(core_axis_name="c", subcore_axis_name="s")
    params = pltpu.CompilerParams(needs_layout_passes=False)

    dots_k = pl.kernel(
        functools.partial(_dots_body, rows_per_worker=rpw,
                          num_cores=info.num_cores),
        out_type=jax.ShapeDtypeStruct((_B,), jnp.float32),
        mesh=mesh,
        compiler_params=params,
        scratch_types=[
            pltpu.VMEM((_NBUF, _C), jnp.int32),
            pltpu.VMEM((_NBUF, _C), jnp.int32),
            pltpu.VMEM((_NBUF, _C, _F), jnp.float32),
            pltpu.VMEM((_NBUF, _C, _F), jnp.float32),
            pltpu.VMEM((_C,), jnp.float32),
            pltpu.SemaphoreType.DMA((_NBUF,)),
        ],
    )
    bias_k = pl.kernel(
        functools.partial(_bias_body, rows_per_worker=rpw,
                          num_cores=info.num_cores),
        out_type=jax.ShapeDtypeStruct((_B,), jnp.float32),
        mesh=mesh,
        compiler_params=params,
        scratch_types=[
            pltpu.VMEM((_NBUF, _C), jnp.int32),
            pltpu.VMEM((_NBUF, _C), jnp.int32),
            pltpu.VMEM((_NBUF, _C), jnp.float32),
            pltpu.VMEM((_NBUF, _C), jnp.float32),
            pltpu.VMEM((_NBUF, _C), jnp.float32),
            pltpu.SemaphoreType.DMA((_NBUF,)),
            pltpu.SemaphoreType.DMA((_NBUF,)),
        ],
    )
    return dots_k, bias_k


def kernel(u_idx, i_idx, u_emb, i_emb, u_bias, i_bias):
    dots_k, bias_k = _build()
    ui = u_idx.astype(jnp.int32)
    ii = i_idx.astype(jnp.int32)
    dots = dots_k(ui, ii, u_emb, i_emb)
    return bias_k(ui, ii, dots, u_bias.reshape(-1), i_bias.reshape(-1))
